# baseline (speedup 1.0000x reference)
import functools

import jax
import jax.numpy as jnp
from jax import lax
from jax.experimental import pallas as pl
from jax.experimental.pallas import tpu as pltpu
from jax.experimental.pallas import tpu_sc as plsc

_B = 16384
_D = 64
_NEG = 20
_NC = 2
_NS = 16
_NW = _NC * _NS
_BW = _B // _NW
_C = 16
_NCH = _BW // _C
_SEG = _C * (1 + _NEG)
_NSCORE = _B * (1 + _NEG)


def _sc_scores(pos_u, pos_v, neg_r, U, V):
    mesh = plsc.VectorSubcoreMesh(core_axis_name="c", subcore_axis_name="s")

    @functools.partial(
        pl.kernel,
        out_type=jax.ShapeDtypeStruct((_NSCORE,), jnp.float32),
        mesh=mesh,
        compiler_params=pltpu.CompilerParams(needs_layout_passes=False,
                                             use_tc_tiling_on_sc=False),
        scratch_types=[
            pltpu.VMEM((_BW,), jnp.int32),
            pltpu.VMEM((_BW,), jnp.int32),
            pltpu.VMEM((_BW * _NEG,), jnp.int32),
            pltpu.VMEM((2 * _C,), jnp.int32),
            pltpu.VMEM((2 * _C,), jnp.int32),
            pltpu.VMEM((2 * _NEG * _C,), jnp.int32),
            pltpu.VMEM((2 * _C, 2 * _D), jnp.float32),
            pltpu.VMEM((2 * _C, 2 * _D), jnp.float32),
            pltpu.VMEM((2 * _NEG * _C, 2 * _D), jnp.float32),
            pltpu.VMEM((_BW * (1 + _NEG) + 16,), jnp.float32),
            pltpu.SemaphoreType.DMA,
            pltpu.SemaphoreType.DMA,
        ],
    )
    def k(pos_u_h, pos_v_h, neg_r_h, U_h, V_h, out_h,
          idx_u, idx_v, idx_n, idx_u_g, idx_v_g, idx_n_g,
          u_rows, v_rows, n_rows, sc_all, sem_a, sem_b):
        wid = lax.axis_index("s") * _NC + lax.axis_index("c")
        li = lax.broadcasted_iota(jnp.int32, (16,), 0)
        base = wid * _BW

        pltpu.sync_copy(pos_u_h.at[pl.ds(base, _BW)], idx_u)
        pltpu.sync_copy(pos_v_h.at[pl.ds(base, _BW)], idx_v)
        pltpu.sync_copy(neg_r_h.at[pl.ds(base * _NEG, _BW * _NEG)], idx_n)

        def _splat(x):
            return jnp.full((16,), x, jnp.int32)

        def _line(v):
            return ((v >> 16) << 14) | (v & 16383)

        def _copies(c, buf, sem):
            nb, cb = buf * _NEG * _C, buf * _C
            cps = [
                pltpu.make_async_copy(
                    U_h.at[idx_u_g.at[pl.ds(cb, _C)]],
                    u_rows.at[pl.ds(cb, _C)], sem),
                pltpu.make_async_copy(
                    V_h.at[idx_v_g.at[pl.ds(cb, _C)]],
                    v_rows.at[pl.ds(cb, _C)], sem),
            ]
            for j in range(_NEG):
                cps.append(pltpu.make_async_copy(
                    V_h.at[idx_n_g.at[pl.ds(nb + j * _C, _C)]],
                    n_rows.at[pl.ds(nb + j * _C, _C)], sem))
            return cps

        def _fire(c, buf, sem):
            nb, cb = buf * _NEG * _C, buf * _C
            idx_u_g[pl.ds(cb, _C)] = _line(idx_u[pl.ds(c * _C, _C)])
            idx_v_g[pl.ds(cb, _C)] = _line(idx_v[pl.ds(c * _C, _C)])

            def sh(i, _):
                idx_n_g[pl.ds(nb + i * 16, 16)] = _line(
                    idx_n[pl.ds(c * _NEG * _C + i * 16, 16)])
                return 0

            lax.fori_loop(0, _NEG * _C // 16, sh, 0)
            for cp in _copies(c, buf, sem):
                cp.start()

        def _wait(c, buf, sem):
            for cp in _copies(c, buf, sem):
                cp.wait()

        def _row(ref, r, pref, rp):
            pv = ((plsc.load_gather(pref, [_splat(rp)]) >> 14) & 3) * 32 + li
            out = []
            for kq in range(2):
                w = plsc.load_gather(ref, [_splat(r), pv + 16 * kq])
                a, b = plsc.unpack(plsc.bitcast(w, jnp.bfloat16),
                                   format=plsc.PackFormat.INTERLEAVED)
                out += [a, b]
            return out

        def _compute(c, buf):
            nb, cb = buf * _NEG * _C, buf * _C
            obase = c * _SEG

            acc = jnp.zeros((16,), jnp.float32)
            for kk in range(16):
                uu = _row(u_rows, cb + kk, idx_u, c * _C + kk)
                vv = _row(v_rows, cb + kk, idx_v, c * _C + kk)
                p = (uu[0] * vv[0] + uu[1] * vv[1]
                     + uu[2] * vv[2] + uu[3] * vv[3])
                acc = jnp.where(li == kk, jnp.sum(p), acc)
            sc_all[pl.ds(obase, 16)] = acc

            def neg_body(b, _):
                u0, u1, u2, u3 = _row(u_rows, cb + b, idx_u, c * _C + b)
                acc1 = jnp.zeros((16,), jnp.float32)
                acc2 = jnp.zeros((16,), jnp.float32)
                for j in range(_NEG):
                    nn = _row(n_rows, nb + j * _C + b, idx_n,
                              c * _NEG * _C + j * _C + b)
                    p = (nn[0] * u0 + nn[1] * u1
                         + nn[2] * u2 + nn[3] * u3)
                    sj = -jnp.sum(p)
                    if j < 16:
                        acc1 = jnp.where(li == j, sj, acc1)
                    else:
                        acc2 = jnp.where(li == (j - 16), sj, acc2)
                ob = obase + _C + b * _NEG
                sc_all[pl.ds(ob, 16)] = acc1
                tail = sc_all[pl.ds(ob + 16, 16)]
                sc_all[pl.ds(ob + 16, 16)] = jnp.where(li < 4, acc2, tail)
                return 0

            lax.fori_loop(0, _C, neg_body, 0)

        _fire(0, 0, sem_a)

        def pair_body(h, _):
            c0 = 2 * h
            _fire(c0 + 1, 1, sem_b)
            _wait(c0, 0, sem_a)
            _compute(c0, 0)

            @pl.when(c0 + 2 < _NCH)
            def _():
                _fire(c0 + 2, 0, sem_a)

            _wait(c0 + 1, 1, sem_b)
            _compute(c0 + 1, 1)
            return 0

        lax.fori_loop(0, _NCH // 2, pair_body, 0)
        pltpu.sync_copy(sc_all.at[pl.ds(0, _BW * (1 + _NEG))],
                        out_h.at[pl.ds(base * (1 + _NEG), _BW * (1 + _NEG))])

    return k(pos_u, pos_v, neg_r, U, V)


def _tc_transpose(xt):
    d, n = xt.shape
    bn = 65536

    def body(x_ref, o_ref):
        x = x_ref[...]
        q4 = bn // 4
        xab = jnp.concatenate(
            [x[:, 0:q4], x[:, q4:2 * q4], x[:, 2 * q4:3 * q4], x[:, 3 * q4:]],
            axis=0)
        jj = lax.broadcasted_iota(jnp.int32, (4 * d, 2 * d), 0)
        cc = lax.broadcasted_iota(jnp.int32, (4 * d, 2 * d), 1)
        base = (cc // 32) * 64 + (cc % 32)
        p_lo = (jj == base).astype(jnp.float32)
        p_hi = (jj == base + 32).astype(jnp.float32)
        dn = (((0,), (0,)), ((), ()))
        lo = lax.dot_general(xab, p_lo, dn, preferred_element_type=jnp.float32)
        hi = lax.dot_general(xab, p_hi, dn, preferred_element_type=jnp.float32)
        lo16 = lax.bitcast_convert_type(lo.astype(jnp.bfloat16),
                                        jnp.uint16).astype(jnp.uint32)
        hi16 = lax.bitcast_convert_type(hi.astype(jnp.bfloat16),
                                        jnp.uint16).astype(jnp.uint32)
        o_ref[...] = lax.bitcast_convert_type((hi16 << 16) | lo16,
                                              jnp.float32)

    nb = pl.cdiv(n, bn)
    return pl.pallas_call(
        body,
        grid=(nb,),
        in_specs=[pl.BlockSpec((d, bn), lambda i: (0, i))],
        out_specs=pl.BlockSpec((bn // 4, 2 * d), lambda i: (i, 0)),
        out_shape=jax.ShapeDtypeStruct((nb * bn // 4, 2 * d), jnp.float32),
    )(xt)


def _tc_logsig_sum(x2d):
    def body(x_ref, o_ref):
        x = x_ref[...]
        ls = jnp.minimum(x, 0.0) - jnp.log1p(jnp.exp(-jnp.abs(x)))
        o_ref[0, 0] = -jnp.sum(ls)

    return pl.pallas_call(
        body,
        out_shape=jax.ShapeDtypeStruct((1, 1), jnp.float32),
        out_specs=pl.BlockSpec(memory_space=pltpu.SMEM),
    )(x2d)


def kernel(pos_u, pos_v, neg_v, U, V):
    neg_r = neg_v.reshape(_NW * _NCH, _C, _NEG).transpose(0, 2, 1).reshape(-1)
    U_rm = _tc_transpose(U.T)
    V_rm = _tc_transpose(V.T)
    scores = _sc_scores(pos_u, pos_v, neg_r, U_rm, V_rm)
    res = _tc_logsig_sum(scores.reshape(_NSCORE // 128, 128))
    return res[0, 0]


# speedup vs baseline: 1.0058x; 1.0058x over previous
import functools

import jax
import jax.numpy as jnp
from jax import lax
from jax.experimental import pallas as pl
from jax.experimental.pallas import tpu as pltpu
from jax.experimental.pallas import tpu_sc as plsc

_B = 16384
_D = 64
_NEG = 20
_NC = 2
_NS = 16
_NW = _NC * _NS
_BW = _B // _NW
_C = 16
_NCH = _BW // _C
_SEG = _C * (1 + _NEG)
_NSCORE = _B * (1 + _NEG)


def _sc_scores(pos_u, pos_v, neg_r, U, V):
    mesh = plsc.VectorSubcoreMesh(core_axis_name="c", subcore_axis_name="s")

    @functools.partial(
        pl.kernel,
        out_type=jax.ShapeDtypeStruct((_NSCORE,), jnp.float32),
        mesh=mesh,
        compiler_params=pltpu.CompilerParams(needs_layout_passes=False,
                                             use_tc_tiling_on_sc=False),
        scratch_types=[
            pltpu.VMEM((_BW,), jnp.int32),
            pltpu.VMEM((_BW,), jnp.int32),
            pltpu.VMEM((_BW * _NEG,), jnp.int32),
            pltpu.VMEM((2 * _C,), jnp.int32),
            pltpu.VMEM((2 * _C,), jnp.int32),
            pltpu.VMEM((2 * _NEG * _C,), jnp.int32),
            pltpu.VMEM((2 * _C, 2 * _D), jnp.float32),
            pltpu.VMEM((2 * _C, 2 * _D), jnp.float32),
            pltpu.VMEM((2 * _NEG * _C, 2 * _D), jnp.float32),
            pltpu.VMEM((_BW * (1 + _NEG) + 16,), jnp.float32),
            pltpu.SemaphoreType.DMA,
            pltpu.SemaphoreType.DMA,
        ],
    )
    def k(pos_u_h, pos_v_h, neg_r_h, U_h, V_h, out_h,
          idx_u, idx_v, idx_n, idx_u_g, idx_v_g, idx_n_g,
          u_rows, v_rows, n_rows, sc_all, sem_a, sem_b):
        wid = lax.axis_index("s") * _NC + lax.axis_index("c")
        li = lax.broadcasted_iota(jnp.int32, (16,), 0)
        base = wid * _BW

        pltpu.sync_copy(pos_u_h.at[pl.ds(base, _BW)], idx_u)
        pltpu.sync_copy(pos_v_h.at[pl.ds(base, _BW)], idx_v)
        pltpu.sync_copy(neg_r_h.at[pl.ds(base * _NEG, _BW * _NEG)], idx_n)

        def _splat(x):
            return jnp.full((16,), x, jnp.int32)

        def _line(v):
            return ((v >> 15) << 13) | (v & 8191)

        def _copies(c, buf, sem):
            nb, cb = buf * _NEG * _C, buf * _C
            cps = [
                pltpu.make_async_copy(
                    U_h.at[idx_u_g.at[pl.ds(cb, _C)]],
                    u_rows.at[pl.ds(cb, _C)], sem),
                pltpu.make_async_copy(
                    V_h.at[idx_v_g.at[pl.ds(cb, _C)]],
                    v_rows.at[pl.ds(cb, _C)], sem),
            ]
            for j in range(_NEG):
                cps.append(pltpu.make_async_copy(
                    V_h.at[idx_n_g.at[pl.ds(nb + j * _C, _C)]],
                    n_rows.at[pl.ds(nb + j * _C, _C)], sem))
            return cps

        def _fire(c, buf, sem):
            nb, cb = buf * _NEG * _C, buf * _C
            idx_u_g[pl.ds(cb, _C)] = _line(idx_u[pl.ds(c * _C, _C)])
            idx_v_g[pl.ds(cb, _C)] = _line(idx_v[pl.ds(c * _C, _C)])

            def sh(i, _):
                idx_n_g[pl.ds(nb + i * 16, 16)] = _line(
                    idx_n[pl.ds(c * _NEG * _C + i * 16, 16)])
                return 0

            lax.fori_loop(0, _NEG * _C // 16, sh, 0)
            for cp in _copies(c, buf, sem):
                cp.start()

        def _wait(c, buf, sem):
            for cp in _copies(c, buf, sem):
                cp.wait()

        def _row(ref, r, pref, rp):
            pv = ((plsc.load_gather(pref, [_splat(rp)]) >> 13) & 3) * 32 + li
            out = []
            for kq in range(2):
                w = plsc.load_gather(ref, [_splat(r), pv + 16 * kq])
                a, b = plsc.unpack(plsc.bitcast(w, jnp.bfloat16),
                                   format=plsc.PackFormat.INTERLEAVED)
                out += [a, b]
            return out

        def _compute(c, buf):
            nb, cb = buf * _NEG * _C, buf * _C
            obase = c * _SEG

            acc = jnp.zeros((16,), jnp.float32)
            for kk in range(16):
                uu = _row(u_rows, cb + kk, idx_u, c * _C + kk)
                vv = _row(v_rows, cb + kk, idx_v, c * _C + kk)
                p = (uu[0] * vv[0] + uu[1] * vv[1]
                     + uu[2] * vv[2] + uu[3] * vv[3])
                acc = jnp.where(li == kk, jnp.sum(p), acc)
            sc_all[pl.ds(obase, 16)] = acc

            def neg_body(b, _):
                u0, u1, u2, u3 = _row(u_rows, cb + b, idx_u, c * _C + b)
                acc1 = jnp.zeros((16,), jnp.float32)
                acc2 = jnp.zeros((16,), jnp.float32)
                for j in range(_NEG):
                    nn = _row(n_rows, nb + j * _C + b, idx_n,
                              c * _NEG * _C + j * _C + b)
                    p = (nn[0] * u0 + nn[1] * u1
                         + nn[2] * u2 + nn[3] * u3)
                    sj = -jnp.sum(p)
                    if j < 16:
                        acc1 = jnp.where(li == j, sj, acc1)
                    else:
                        acc2 = jnp.where(li == (j - 16), sj, acc2)
                ob = obase + _C + b * _NEG
                sc_all[pl.ds(ob, 16)] = acc1
                tail = sc_all[pl.ds(ob + 16, 16)]
                sc_all[pl.ds(ob + 16, 16)] = jnp.where(li < 4, acc2, tail)
                return 0

            lax.fori_loop(0, _C, neg_body, 0)

        _fire(0, 0, sem_a)

        def pair_body(h, _):
            c0 = 2 * h
            _fire(c0 + 1, 1, sem_b)
            _wait(c0, 0, sem_a)
            _compute(c0, 0)

            @pl.when(c0 + 2 < _NCH)
            def _():
                _fire(c0 + 2, 0, sem_a)

            _wait(c0 + 1, 1, sem_b)
            _compute(c0 + 1, 1)
            return 0

        lax.fori_loop(0, _NCH // 2, pair_body, 0)
        pltpu.sync_copy(sc_all.at[pl.ds(0, _BW * (1 + _NEG))],
                        out_h.at[pl.ds(base * (1 + _NEG), _BW * (1 + _NEG))])

    return k(pos_u, pos_v, neg_r, U, V)


def _tc_transpose(xt):
    d, n = xt.shape
    bn = 32768

    def body(x_ref, o_ref):
        x = x_ref[...]
        q4 = bn // 4
        xab = jnp.concatenate(
            [x[:, 0:q4], x[:, q4:2 * q4], x[:, 2 * q4:3 * q4], x[:, 3 * q4:]],
            axis=0)
        jj = lax.broadcasted_iota(jnp.int32, (4 * d, 2 * d), 0)
        cc = lax.broadcasted_iota(jnp.int32, (4 * d, 2 * d), 1)
        base = (cc // 32) * 64 + (cc % 32)
        p_lo = (jj == base).astype(jnp.float32)
        p_hi = (jj == base + 32).astype(jnp.float32)
        dn = (((0,), (0,)), ((), ()))
        lo = lax.dot_general(xab, p_lo, dn, preferred_element_type=jnp.float32)
        hi = lax.dot_general(xab, p_hi, dn, preferred_element_type=jnp.float32)
        lo16 = lax.bitcast_convert_type(lo.astype(jnp.bfloat16),
                                        jnp.uint16).astype(jnp.uint32)
        hi16 = lax.bitcast_convert_type(hi.astype(jnp.bfloat16),
                                        jnp.uint16).astype(jnp.uint32)
        o_ref[...] = lax.bitcast_convert_type((hi16 << 16) | lo16,
                                              jnp.float32)

    nb = pl.cdiv(n, bn)
    return pl.pallas_call(
        body,
        grid=(nb,),
        in_specs=[pl.BlockSpec((d, bn), lambda i: (0, i))],
        out_specs=pl.BlockSpec((bn // 4, 2 * d), lambda i: (i, 0)),
        out_shape=jax.ShapeDtypeStruct((nb * bn // 4, 2 * d), jnp.float32),
    )(xt)


def _tc_logsig_sum(x2d):
    def body(x_ref, o_ref):
        x = x_ref[...]
        ls = jnp.minimum(x, 0.0) - jnp.log1p(jnp.exp(-jnp.abs(x)))
        o_ref[0, 0] = -jnp.sum(ls)

    return pl.pallas_call(
        body,
        out_shape=jax.ShapeDtypeStruct((1, 1), jnp.float32),
        out_specs=pl.BlockSpec(memory_space=pltpu.SMEM),
    )(x2d)


def kernel(pos_u, pos_v, neg_v, U, V):
    neg_r = neg_v.reshape(_NW * _NCH, _C, _NEG).transpose(0, 2, 1).reshape(-1)
    U_rm = _tc_transpose(U.T)
    V_rm = _tc_transpose(V.T)
    scores = _sc_scores(pos_u, pos_v, neg_r, U_rm, V_rm)
    res = _tc_logsig_sum(scores.reshape(_NSCORE // 128, 128))
    return res[0, 0]
